# baseline (speedup 1.0000x reference)
import jax
import jax.numpy as jnp
from jax.experimental import pallas as pl
from jax.experimental.pallas import tpu as pltpu


def _make_outer(bm, bs, nsup, n, nfeat, nhid, nclass, buf):
    def outer(adj_hbm, x_hbm, w1_ref, b1_ref, w2_ref, b2_ref,
              gc1_hbm, out_hbm, sup_ref):
        def body(idx, x_blk, adj_blk, gc1_blk, out_blk):
            (i,) = idx

            @pl.when(i < nsup)
            def _():
                sup_ref[pl.ds(pl.multiple_of(i * bs, 16), bs), :] = jnp.dot(
                    x_blk[...].astype(jnp.bfloat16),
                    w1_ref[...].astype(jnp.bfloat16),
                    preferred_element_type=jnp.float32,
                ).astype(jnp.bfloat16)

            @pl.when(i >= nsup)
            def _():
                g = jax.lax.dot_general(
                    adj_blk[...], sup_ref[...],
                    (((1,), (0,)), ((), ())),
                    preferred_element_type=jnp.float32)
                g = jnp.maximum(g + b1_ref[...], 0.0)
                gc1_blk[...] = g
                w2 = w2_ref[...].astype(jnp.bfloat16)
                logits = jax.lax.dot_general(
                    g.astype(jnp.bfloat16), w2,
                    (((1,), (1,)), ((), ())),
                    preferred_element_type=jnp.float32,
                ) + b2_ref[...]
                mx = jnp.max(logits, axis=1, keepdims=True)
                e = jnp.exp(logits - mx)
                out_blk[...] = e / jnp.sum(e, axis=1, keepdims=True)

        pltpu.emit_pipeline(
            body,
            grid=(nsup + n // bm,),
            in_specs=[
                pl.BlockSpec((bs, nfeat),
                             lambda i: (jnp.minimum(i, nsup - 1), 0)),
                pl.BlockSpec((bm, n),
                             lambda i: (jnp.maximum(i - nsup, 0), 0),
                             pipeline_mode=pl.Buffered(
                                 buffer_count=buf, use_lookahead=True)),
            ],
            out_specs=[
                pl.BlockSpec((bm, nhid),
                             lambda i: (jnp.maximum(i - nsup, 0), 0)),
                pl.BlockSpec((bm, nclass),
                             lambda i: (jnp.maximum(i - nsup, 0), 0)),
            ],
            _explicit_indices=True,
        )(x_hbm, adj_hbm, gc1_hbm, out_hbm)

    return outer


def kernel(x, adj, gc1_weight, gc1_bias, fc2_weight, fc2_bias):
    n, nfeat = x.shape
    nhid = gc1_weight.shape[1]
    nclass = fc2_weight.shape[0]

    bm = 200 if n % 200 == 0 else n
    nsup = 5 if n % 80 == 0 else 1
    bs = n // nsup
    b1 = gc1_bias.reshape(1, nhid)
    b2 = fc2_bias.reshape(1, nclass)
    gc1, out = pl.pallas_call(
        _make_outer(bm, bs, nsup, n, nfeat, nhid, nclass, buf=3),
        in_specs=[
            pl.BlockSpec(memory_space=pl.ANY),
            pl.BlockSpec(memory_space=pl.ANY),
            pl.BlockSpec((nfeat, nhid), lambda: (0, 0)),
            pl.BlockSpec((1, nhid), lambda: (0, 0)),
            pl.BlockSpec((nclass, nhid), lambda: (0, 0)),
            pl.BlockSpec((1, nclass), lambda: (0, 0)),
        ],
        out_specs=[
            pl.BlockSpec(memory_space=pl.ANY),
            pl.BlockSpec(memory_space=pl.ANY),
        ],
        out_shape=[
            jax.ShapeDtypeStruct((n, nhid), jnp.float32),
            jax.ShapeDtypeStruct((n, nclass), jnp.float32),
        ],
        scratch_shapes=[pltpu.VMEM((n, nhid), jnp.bfloat16)],
        compiler_params=pltpu.CompilerParams(
            vmem_limit_bytes=60 * 1024 * 1024),
    )(adj, x, gc1_weight, b1, fc2_weight, b2)

    return (gc1, out)


# speedup vs baseline: 1.0274x; 1.0274x over previous
import jax
import jax.numpy as jnp
from jax.experimental import pallas as pl
from jax.experimental.pallas import tpu as pltpu


def _make_outer(bm, bs, n, nfeat, nhid, nclass, buf):
    def outer(adj_hbm, x_hbm, w1_ref, b1_ref, w2_ref, b2_ref,
              gc1_hbm, out_hbm, sup_ref):
        def sup_body(idx, x_blk):
            (i,) = idx
            sup_ref[pl.ds(pl.multiple_of(i * bs, 16), bs), :] = jnp.dot(
                x_blk[...].astype(jnp.bfloat16),
                w1_ref[...].astype(jnp.bfloat16),
                preferred_element_type=jnp.float32,
            ).astype(jnp.bfloat16)

        def main_body(adj_blk, gc1_blk, out_blk):
            g = jax.lax.dot_general(
                adj_blk[...], sup_ref[...],
                (((1,), (0,)), ((), ())),
                preferred_element_type=jnp.float32)
            g = jnp.maximum(g + b1_ref[...], 0.0)
            gc1_blk[...] = g
            w2 = w2_ref[...].astype(jnp.bfloat16)
            logits = jax.lax.dot_general(
                g.astype(jnp.bfloat16), w2,
                (((1,), (1,)), ((), ())),
                preferred_element_type=jnp.float32,
            ) + b2_ref[...]
            mx = jnp.max(logits, axis=1, keepdims=True)
            e = jnp.exp(logits - mx)
            out_blk[...] = e / jnp.sum(e, axis=1, keepdims=True)

        pltpu.emit_pipeline(
            sup_body,
            grid=(n // bs,),
            in_specs=[pl.BlockSpec((bs, nfeat), lambda i: (i, 0))],
            _explicit_indices=True,
        )(x_hbm)

        pltpu.emit_pipeline(
            main_body,
            grid=(n // bm,),
            in_specs=[
                pl.BlockSpec((bm, n), lambda i: (i, 0),
                             pipeline_mode=pl.Buffered(buffer_count=buf)),
            ],
            out_specs=[
                pl.BlockSpec((bm, nhid), lambda i: (i, 0)),
                pl.BlockSpec((bm, nclass), lambda i: (i, 0)),
            ],
        )(adj_hbm, gc1_hbm, out_hbm)

    return outer


def kernel(x, adj, gc1_weight, gc1_bias, fc2_weight, fc2_bias):
    n, nfeat = x.shape
    nhid = gc1_weight.shape[1]
    nclass = fc2_weight.shape[0]

    bm = 400 if n % 400 == 0 else n
    bs = 1000 if n % 1000 == 0 else n
    b1 = gc1_bias.reshape(1, nhid)
    b2 = fc2_bias.reshape(1, nclass)
    gc1, out = pl.pallas_call(
        _make_outer(bm, bs, n, nfeat, nhid, nclass, buf=2),
        in_specs=[
            pl.BlockSpec(memory_space=pl.ANY),
            pl.BlockSpec(memory_space=pl.ANY),
            pl.BlockSpec((nfeat, nhid), lambda: (0, 0)),
            pl.BlockSpec((1, nhid), lambda: (0, 0)),
            pl.BlockSpec((nclass, nhid), lambda: (0, 0)),
            pl.BlockSpec((1, nclass), lambda: (0, 0)),
        ],
        out_specs=[
            pl.BlockSpec(memory_space=pl.ANY),
            pl.BlockSpec(memory_space=pl.ANY),
        ],
        out_shape=[
            jax.ShapeDtypeStruct((n, nhid), jnp.float32),
            jax.ShapeDtypeStruct((n, nclass), jnp.float32),
        ],
        scratch_shapes=[pltpu.VMEM((n, nhid), jnp.bfloat16)],
        compiler_params=pltpu.CompilerParams(
            vmem_limit_bytes=60 * 1024 * 1024),
    )(adj, x, gc1_weight, b1, fc2_weight, b2)

    return (gc1, out)


# speedup vs baseline: 1.0319x; 1.0045x over previous
import jax
import jax.numpy as jnp
from jax.experimental import pallas as pl
from jax.experimental.pallas import tpu as pltpu


def _make_outer(bm, bs, n, nfeat, nhid, nclass, buf):
    def outer(adj_hbm, x_hbm, w1_ref, b1_ref, w2_ref, b2_ref,
              gc1_hbm, out_hbm, sup_ref):
        def sup_body(idx, x_blk):
            (i,) = idx
            sup_ref[pl.ds(pl.multiple_of(i * bs, 16), bs), :] = jnp.dot(
                x_blk[...].astype(jnp.bfloat16),
                w1_ref[...].astype(jnp.bfloat16),
                preferred_element_type=jnp.float32,
            ).astype(jnp.bfloat16)

        def main_body(adj_blk, gc1_blk, out_blk):
            g = jax.lax.dot_general(
                adj_blk[...], sup_ref[...],
                (((1,), (0,)), ((), ())),
                preferred_element_type=jnp.float32)
            g = jnp.maximum(g + b1_ref[...], 0.0)
            gc1_blk[...] = g
            w2 = w2_ref[...].astype(jnp.bfloat16)
            logits = jax.lax.dot_general(
                g.astype(jnp.bfloat16), w2,
                (((1,), (1,)), ((), ())),
                preferred_element_type=jnp.float32,
            ) + b2_ref[...]
            mx = jnp.max(logits, axis=1, keepdims=True)
            e = jnp.exp(logits - mx)
            out_blk[...] = e / jnp.sum(e, axis=1, keepdims=True)

        pltpu.emit_pipeline(
            sup_body,
            grid=(n // bs,),
            in_specs=[pl.BlockSpec((bs, nfeat), lambda i: (i, 0))],
            _explicit_indices=True,
        )(x_hbm)

        pltpu.emit_pipeline(
            main_body,
            grid=(n // bm,),
            in_specs=[
                pl.BlockSpec((bm, n), lambda i: (i, 0),
                             pipeline_mode=pl.Buffered(buffer_count=buf)),
            ],
            out_specs=[
                pl.BlockSpec((bm, nhid), lambda i: (i, 0)),
                pl.BlockSpec((bm, nclass), lambda i: (i, 0)),
            ],
        )(adj_hbm, gc1_hbm, out_hbm)

    return outer


def kernel(x, adj, gc1_weight, gc1_bias, fc2_weight, fc2_bias):
    n, nfeat = x.shape
    nhid = gc1_weight.shape[1]
    nclass = fc2_weight.shape[0]

    bm = 200 if n % 200 == 0 else n
    bs = 1000 if n % 1000 == 0 else n
    b1 = gc1_bias.reshape(1, nhid)
    b2 = fc2_bias.reshape(1, nclass)
    gc1, out = pl.pallas_call(
        _make_outer(bm, bs, n, nfeat, nhid, nclass, buf=4),
        in_specs=[
            pl.BlockSpec(memory_space=pl.ANY),
            pl.BlockSpec(memory_space=pl.ANY),
            pl.BlockSpec((nfeat, nhid), lambda: (0, 0)),
            pl.BlockSpec((1, nhid), lambda: (0, 0)),
            pl.BlockSpec((nclass, nhid), lambda: (0, 0)),
            pl.BlockSpec((1, nclass), lambda: (0, 0)),
        ],
        out_specs=[
            pl.BlockSpec(memory_space=pl.ANY),
            pl.BlockSpec(memory_space=pl.ANY),
        ],
        out_shape=[
            jax.ShapeDtypeStruct((n, nhid), jnp.float32),
            jax.ShapeDtypeStruct((n, nclass), jnp.float32),
        ],
        scratch_shapes=[pltpu.VMEM((n, nhid), jnp.bfloat16)],
        compiler_params=pltpu.CompilerParams(
            vmem_limit_bytes=60 * 1024 * 1024),
    )(adj, x, gc1_weight, b1, fc2_weight, b2)

    return (gc1, out)


# speedup vs baseline: 1.0347x; 1.0027x over previous
import jax
import jax.numpy as jnp
from jax.experimental import pallas as pl
from jax.experimental.pallas import tpu as pltpu


def _make_outer(bm, bs, n, nfeat, nhid, nclass, buf):
    def outer(adj_hbm, x_hbm, w1_ref, b1_ref, w2_ref, b2_ref,
              gc1_hbm, out_hbm, sup_ref, w2b_ref):
        w2b_ref[...] = w2_ref[...].astype(jnp.bfloat16)
        def sup_body(idx, x_blk):
            (i,) = idx
            sup_ref[pl.ds(pl.multiple_of(i * bs, 16), bs), :] = jnp.dot(
                x_blk[...].astype(jnp.bfloat16),
                w1_ref[...].astype(jnp.bfloat16),
                preferred_element_type=jnp.float32,
            ).astype(jnp.bfloat16)

        def main_body(adj_blk, gc1_blk, out_blk):
            g = jax.lax.dot_general(
                adj_blk[...], sup_ref[...],
                (((1,), (0,)), ((), ())),
                preferred_element_type=jnp.float32)
            g = jnp.maximum(g + b1_ref[...], 0.0)
            gc1_blk[...] = g
            logits = jax.lax.dot_general(
                g.astype(jnp.bfloat16), w2b_ref[...],
                (((1,), (1,)), ((), ())),
                preferred_element_type=jnp.float32,
            ) + b2_ref[...]
            mx = jnp.max(logits, axis=1, keepdims=True)
            e = jnp.exp(logits - mx)
            out_blk[...] = e / jnp.sum(e, axis=1, keepdims=True)

        pltpu.emit_pipeline(
            sup_body,
            grid=(n // bs,),
            in_specs=[pl.BlockSpec((bs, nfeat), lambda i: (i, 0))],
            _explicit_indices=True,
        )(x_hbm)

        pltpu.emit_pipeline(
            main_body,
            grid=(n // bm,),
            in_specs=[
                pl.BlockSpec((bm, n), lambda i: (i, 0),
                             pipeline_mode=pl.Buffered(buffer_count=buf)),
            ],
            out_specs=[
                pl.BlockSpec((bm, nhid), lambda i: (i, 0)),
                pl.BlockSpec((bm, nclass), lambda i: (i, 0)),
            ],
        )(adj_hbm, gc1_hbm, out_hbm)

    return outer


def kernel(x, adj, gc1_weight, gc1_bias, fc2_weight, fc2_bias):
    n, nfeat = x.shape
    nhid = gc1_weight.shape[1]
    nclass = fc2_weight.shape[0]

    bm = 200 if n % 200 == 0 else n
    bs = 1000 if n % 1000 == 0 else n
    b1 = gc1_bias.reshape(1, nhid)
    b2 = fc2_bias.reshape(1, nclass)
    gc1, out = pl.pallas_call(
        _make_outer(bm, bs, n, nfeat, nhid, nclass, buf=4),
        in_specs=[
            pl.BlockSpec(memory_space=pl.ANY),
            pl.BlockSpec(memory_space=pl.ANY),
            pl.BlockSpec((nfeat, nhid), lambda: (0, 0)),
            pl.BlockSpec((1, nhid), lambda: (0, 0)),
            pl.BlockSpec((nclass, nhid), lambda: (0, 0)),
            pl.BlockSpec((1, nclass), lambda: (0, 0)),
        ],
        out_specs=[
            pl.BlockSpec(memory_space=pl.ANY),
            pl.BlockSpec(memory_space=pl.ANY),
        ],
        out_shape=[
            jax.ShapeDtypeStruct((n, nhid), jnp.float32),
            jax.ShapeDtypeStruct((n, nclass), jnp.float32),
        ],
        scratch_shapes=[pltpu.VMEM((n, nhid), jnp.bfloat16),
                        pltpu.VMEM((nclass, nhid), jnp.bfloat16)],
        compiler_params=pltpu.CompilerParams(
            vmem_limit_bytes=60 * 1024 * 1024),
    )(adj, x, gc1_weight, b1, fc2_weight, b2)

    return (gc1, out)
